# baseline (speedup 1.0000x reference)
import functools

import jax
import jax.numpy as jnp
from jax.experimental import pallas as pl
from jax.experimental.pallas import tpu as pltpu


def _moe_body(x_ref, es_ref, k_ref, v_ref, o_ref):
    j = pl.program_id(1)
    xb = x_ref[...]
    epb = k_ref.shape[0]
    contrib = None
    for u in range(epb):
        g = jax.nn.sigmoid(
            jax.lax.dot_general(
                xb, es_ref[u], (((1,), (1,)), ((), ())),
                preferred_element_type=jnp.float32,
            )
        )
        h = jnp.maximum(
            jnp.dot(xb, k_ref[u], preferred_element_type=jnp.float32), 0.0
        )
        pc = jnp.dot(h * g, v_ref[u], preferred_element_type=jnp.float32)
        contrib = pc if contrib is None else contrib + pc

    @pl.when(j == 0)
    def _init():
        o_ref[...] = contrib

    @pl.when(j != 0)
    def _acc():
        o_ref[...] += contrib


@functools.partial(jax.jit, static_argnames=("tm", "epb"))
def _moe(x, es3, keys, values, tm, epb):
    t, d = x.shape
    e, _, h = keys.shape
    out = pl.pallas_call(
        _moe_body,
        grid=(t // tm, e // epb),
        in_specs=[
            pl.BlockSpec((tm, d), lambda i, j: (i, 0)),
            pl.BlockSpec((epb, 1, d), lambda i, j: (j, 0, 0)),
            pl.BlockSpec((epb, d, h), lambda i, j: (j, 0, 0)),
            pl.BlockSpec((epb, h, d), lambda i, j: (j, 0, 0)),
        ],
        out_specs=pl.BlockSpec((tm, d), lambda i, j: (i, 0)),
        out_shape=jax.ShapeDtypeStruct((t, d), jnp.float32),
        compiler_params=pltpu.CompilerParams(
            dimension_semantics=("parallel", "arbitrary"),
        ),
    )(x, es3, keys, values)
    return out


def kernel(input_tensor, expert_sel, keys, values, bias):
    b, s, d = input_tensor.shape
    n_exp = keys.shape[0]
    x = input_tensor.reshape(b * s, d)
    es3 = expert_sel.reshape(n_exp, 1, d)
    out = _moe(x, es3, keys, values, tm=512, epb=2)
    return out.reshape(b, s, d)


# speedup vs baseline: 1.0685x; 1.0685x over previous
import functools

import jax
import jax.numpy as jnp
from jax.experimental import pallas as pl
from jax.experimental.pallas import tpu as pltpu


def _moe_body(x_ref, es_ref, k_ref, v_ref, o_ref):
    j = pl.program_id(0)
    i = pl.program_id(1)
    tm = x_ref.shape[0]
    xb = x_ref[...]
    epb = k_ref.shape[0]
    contrib = None
    for u in range(epb):
        g = jax.nn.sigmoid(
            jax.lax.dot_general(
                xb, es_ref[u], (((1,), (1,)), ((), ())),
                preferred_element_type=jnp.float32,
            )
        )
        h = jnp.maximum(
            jnp.dot(xb, k_ref[u], preferred_element_type=jnp.float32), 0.0
        )
        pc = jnp.dot(h * g, v_ref[u], preferred_element_type=jnp.float32)
        contrib = pc if contrib is None else contrib + pc

    @pl.when(j == 0)
    def _init():
        o_ref[pl.ds(i * tm, tm), :] = contrib

    @pl.when(j != 0)
    def _acc():
        o_ref[pl.ds(i * tm, tm), :] += contrib


@functools.partial(jax.jit, static_argnames=("tm", "epb"))
def _moe(x, es3, keys, values, tm, epb):
    t, d = x.shape
    e, _, h = keys.shape
    out = pl.pallas_call(
        _moe_body,
        grid=(e // epb, t // tm),
        in_specs=[
            pl.BlockSpec((tm, d), lambda j, i: (i, 0)),
            pl.BlockSpec((epb, 1, d), lambda j, i: (j, 0, 0)),
            pl.BlockSpec((epb, d, h), lambda j, i: (j, 0, 0)),
            pl.BlockSpec((epb, h, d), lambda j, i: (j, 0, 0)),
        ],
        out_specs=pl.BlockSpec((t, d), lambda j, i: (0, 0)),
        out_shape=jax.ShapeDtypeStruct((t, d), jnp.float32),
        compiler_params=pltpu.CompilerParams(
            dimension_semantics=("arbitrary", "arbitrary"),
        ),
    )(x, es3, keys, values)
    return out


def kernel(input_tensor, expert_sel, keys, values, bias):
    b, s, d = input_tensor.shape
    n_exp = keys.shape[0]
    x = input_tensor.reshape(b * s, d)
    es3 = expert_sel.reshape(n_exp, 1, d)
    out = _moe(x, es3, keys, values, tm=512, epb=2)
    return out.reshape(b, s, d)


# speedup vs baseline: 1.1651x; 1.0904x over previous
import functools

import jax
import jax.numpy as jnp
from jax.experimental import pallas as pl
from jax.experimental.pallas import tpu as pltpu


def _moe_body(x_ref, es_ref, k_ref, v_ref, o_ref):
    j = pl.program_id(1)
    xb = x_ref[...]
    epb, _, hdim = k_ref.shape
    gated = []
    for u in range(epb):
        g = jax.nn.sigmoid(
            jax.lax.dot_general(
                xb, es_ref[u], (((1,), (1,)), ((), ())),
                preferred_element_type=jnp.float32,
            )
        )
        h = jnp.maximum(
            jnp.dot(xb, k_ref[u], preferred_element_type=jnp.float32), 0.0
        )
        gated.append(h * g)
    hg = jnp.concatenate(gated, axis=1)
    v2 = v_ref[...].reshape(epb * hdim, v_ref.shape[2])
    contrib = jnp.dot(hg, v2, preferred_element_type=jnp.float32)

    @pl.when(j == 0)
    def _init():
        o_ref[...] = contrib

    @pl.when(j != 0)
    def _acc():
        o_ref[...] += contrib


@functools.partial(jax.jit, static_argnames=("tm", "epb"))
def _moe(x, es3, keys, values, tm, epb):
    t, d = x.shape
    e, _, h = keys.shape
    out = pl.pallas_call(
        _moe_body,
        grid=(t // tm, e // epb),
        in_specs=[
            pl.BlockSpec((tm, d), lambda i, j: (i, 0)),
            pl.BlockSpec((epb, 1, d), lambda i, j: (j, 0, 0)),
            pl.BlockSpec((epb, d, h), lambda i, j: (j, 0, 0)),
            pl.BlockSpec((epb, h, d), lambda i, j: (j, 0, 0)),
        ],
        out_specs=pl.BlockSpec((tm, d), lambda i, j: (i, 0)),
        out_shape=jax.ShapeDtypeStruct((t, d), jnp.float32),
        compiler_params=pltpu.CompilerParams(
            dimension_semantics=("parallel", "arbitrary"),
        ),
    )(x, es3, keys, values)
    return out


def kernel(input_tensor, expert_sel, keys, values, bias):
    b, s, d = input_tensor.shape
    n_exp = keys.shape[0]
    x = input_tensor.reshape(b * s, d)
    es3 = expert_sel.reshape(n_exp, 1, d)
    out = _moe(x, es3, keys, values, tm=1024, epb=2)
    return out.reshape(b, s, d)
